# baseline (speedup 1.0000x reference)
import dataclasses
import functools

import jax
import jax.numpy as jnp
from jax.experimental import pallas as pl
from jax.experimental.pallas import tpu as pltpu
from jax.experimental.pallas import tpu_sc as plsc

ROWS = 16384
COLS = 1000
K = 200

R = 32

_OFFS = tuple(range(0, K - 16, 16)) + (K - 16,)

_mesh = plsc.VectorSubcoreMesh(core_axis_name="c", subcore_axis_name="s")

_cp = pltpu.CompilerParams()
if "needs_layout_passes" in pltpu.CompilerParams.__dataclass_fields__:
    _cp = dataclasses.replace(_cp, needs_layout_passes=False)


@functools.partial(
    pl.kernel,
    mesh=_mesh,
    compiler_params=_cp,
    out_type=jax.ShapeDtypeStruct((ROWS, K), jnp.float32),
)
def _gather_dim1(in_hbm, idx_hbm, out_hbm):
    def body(in_v, idx_v, out_v):
        @pl.loop(0, R, step=2)
        def _(r):
            for rr in range(2):
                rsplat = jnp.full((16,), r + rr, jnp.int32)
                for off in _OFFS:
                    idx16 = idx_v[r + rr, pl.ds(off, 16)]
                    vals = plsc.load_gather(in_v, [rsplat, idx16])
                    out_v[r + rr, pl.ds(off, 16)] = vals

    pltpu.emit_pipeline(
        body,
        grid=(ROWS // R,),
        in_specs=[
            pl.BlockSpec((R, COLS), lambda i: (i, 0)),
            pl.BlockSpec((R, K), lambda i: (i, 0)),
        ],
        out_specs=[pl.BlockSpec((R, K), lambda i: (i, 0))],
        core_axis_name=("c", "s"),
        dimension_semantics=(pltpu.PARALLEL,),
    )(in_hbm, idx_hbm, out_hbm)


def kernel(input, index):
    return _gather_dim1(input, index.astype(jnp.int32))
